# baseline (speedup 1.0000x reference)
import jax
import jax.numpy as jnp
from jax.experimental import pallas as pl


def _body(x_ref, m_ref, o_ref):
    o_ref[...] = x_ref[...] + m_ref[...].astype(jnp.float32)


def kernel(x, mask):
    R, C = x.shape
    BR = 256
    return pl.pallas_call(
        _body,
        grid=(R // BR,),
        in_specs=[
            pl.BlockSpec((BR, C), lambda i: (i, 0)),
            pl.BlockSpec((BR, C), lambda i: (i, 0)),
        ],
        out_specs=pl.BlockSpec((BR, C), lambda i: (i, 0)),
        out_shape=jax.ShapeDtypeStruct((R, C), x.dtype),
    )(x, mask)


# speedup vs baseline: 1.5217x; 1.5217x over previous
import jax
import jax.numpy as jnp
from jax.experimental import pallas as pl


def _body(x_ref, m_ref, o_ref):
    o_ref[...] = x_ref[...] + m_ref[...].astype(jnp.float32)


def kernel(x, mask):
    R, C = x.shape
    BR = 2048
    return pl.pallas_call(
        _body,
        grid=(R // BR,),
        in_specs=[
            pl.BlockSpec((BR, C), lambda i: (i, 0)),
            pl.BlockSpec((BR, C), lambda i: (i, 0)),
        ],
        out_specs=pl.BlockSpec((BR, C), lambda i: (i, 0)),
        out_shape=jax.ShapeDtypeStruct((R, C), x.dtype),
    )(x, mask)


# speedup vs baseline: 2.2886x; 1.5040x over previous
import jax
import jax.numpy as jnp
from jax.experimental import pallas as pl


def _body(x_ref, m_ref, o_ref):
    o_ref[...] = x_ref[...] + m_ref[...].astype(jnp.float32)


def kernel(x, mask):
    R, C = x.shape
    BR = 2048
    m8 = mask.view(jnp.int8)
    return pl.pallas_call(
        _body,
        grid=(R // BR,),
        in_specs=[
            pl.BlockSpec((BR, C), lambda i: (i, 0)),
            pl.BlockSpec((BR, C), lambda i: (i, 0)),
        ],
        out_specs=pl.BlockSpec((BR, C), lambda i: (i, 0)),
        out_shape=jax.ShapeDtypeStruct((R, C), x.dtype),
    )(x, m8)
